# baseline (speedup 1.0000x reference)
import functools

import jax
import jax.numpy as jnp
from jax import lax
from jax.experimental import pallas as pl
from jax.experimental.pallas import tpu as pltpu
from jax.experimental.pallas import tpu_sc as plsc

BATCH = 4096
HIST = 200
EMBED = 64
CHUNK = 100
CHUNKS_PER_ROW = HIST // CHUNK


def _make_pooling_kernel():
  info = plsc.get_sparse_core_info()
  nw = info.num_cores * info.num_subcores
  b_per_w = BATCH // nw
  n_chunks = b_per_w * CHUNKS_PER_ROW

  mesh = plsc.VectorSubcoreMesh(core_axis_name="c", subcore_axis_name="s")

  @functools.partial(
      pl.kernel,
      mesh=mesh,
      compiler_params=pltpu.CompilerParams(use_tc_tiling_on_sc=False),
      out_type=jax.ShapeDtypeStruct((BATCH, EMBED), jnp.float32),
      scratch_types=[
          pltpu.VMEM((n_chunks, CHUNK), jnp.int32),
          pltpu.VMEM((HIST, EMBED), jnp.float32),
          pltpu.VMEM((b_per_w, EMBED), jnp.float32),
          pltpu.SemaphoreType.DMA,
      ],
  )
  def pool(x_hbm, table_hbm, out_hbm, idx_v, rows_v, pooled_v, sem):
    wid = lax.axis_index("s") * info.num_cores + lax.axis_index("c")
    base = wid * b_per_w

    pltpu.sync_copy(x_hbm.at[pl.ds(base * CHUNKS_PER_ROW, n_chunks)], idx_v)

    def row_body(b, _):
      cps = pltpu.async_copy(
          table_hbm.at[idx_v.at[b * 2]], rows_v.at[pl.ds(0, CHUNK)], sem)
      cps2 = pltpu.async_copy(
          table_hbm.at[idx_v.at[b * 2 + 1]], rows_v.at[pl.ds(CHUNK, CHUNK)],
          sem)
      cps.wait()
      cps2.wait()

      def sum_body(l, acc):
        a0, a1, a2, a3 = acc
        a0 = a0 + rows_v[l, pl.ds(0, 16)]
        a1 = a1 + rows_v[l, pl.ds(16, 16)]
        a2 = a2 + rows_v[l, pl.ds(32, 16)]
        a3 = a3 + rows_v[l, pl.ds(48, 16)]
        return (a0, a1, a2, a3)

      zero = jnp.zeros((16,), jnp.float32)
      a0, a1, a2, a3 = lax.fori_loop(
          0, HIST, sum_body, (zero, zero, zero, zero))
      pooled_v[b, pl.ds(0, 16)] = a0
      pooled_v[b, pl.ds(16, 16)] = a1
      pooled_v[b, pl.ds(32, 16)] = a2
      pooled_v[b, pl.ds(48, 16)] = a3
      return ()

    lax.fori_loop(0, b_per_w, row_body, ())

    pltpu.sync_copy(pooled_v, out_hbm.at[pl.ds(base, b_per_w)])

  return pool


_pooling_kernel = _make_pooling_kernel()


def _mlp_kernel(pooled_ref, w1_ref, b1_ref, w2_ref, b2_ref, out_ref):
  h = jnp.dot(pooled_ref[...], w1_ref[...],
              preferred_element_type=jnp.float32)
  h = jnp.maximum(h + b1_ref[...], 0.0)
  out_ref[...] = jnp.dot(h, w2_ref[...],
                         preferred_element_type=jnp.float32) + b2_ref[...]


@jax.jit
def kernel(x, table, W1, b1, W2, b2):
  x2 = x.reshape(BATCH * CHUNKS_PER_ROW, CHUNK)
  pooled = _pooling_kernel(x2, table)
  out = pl.pallas_call(
      _mlp_kernel,
      out_shape=jax.ShapeDtypeStruct((BATCH, 4), jnp.float32),
  )(pooled, W1, b1.reshape(1, 100), W2, b2.reshape(1, 4))
  return out


# speedup vs baseline: 1.2380x; 1.2380x over previous
import functools

import jax
import jax.numpy as jnp
from jax import lax
from jax.experimental import pallas as pl
from jax.experimental.pallas import tpu as pltpu
from jax.experimental.pallas import tpu_sc as plsc

BATCH = 4096
HIST = 200
EMBED = 64
CHUNK = 100
CHUNKS_PER_ROW = HIST // CHUNK


NBUF = 4
UNROLL = 8


def _make_pooling_kernel():
  info = plsc.get_sparse_core_info()
  nw = info.num_cores * info.num_subcores
  b_per_w = BATCH // nw
  n_chunks = b_per_w * CHUNKS_PER_ROW

  mesh = plsc.VectorSubcoreMesh(core_axis_name="c", subcore_axis_name="s")

  @functools.partial(
      pl.kernel,
      mesh=mesh,
      compiler_params=pltpu.CompilerParams(use_tc_tiling_on_sc=False),
      out_type=jax.ShapeDtypeStruct((BATCH, EMBED), jnp.float32),
      scratch_types=[
          pltpu.VMEM((n_chunks, CHUNK), jnp.int32),
          pltpu.VMEM((NBUF, HIST, EMBED), jnp.float32),
          pltpu.VMEM((b_per_w, EMBED), jnp.float32),
          [pltpu.SemaphoreType.DMA] * NBUF,
      ],
  )
  def pool(x_hbm, table_hbm, out_hbm, idx_v, rows_v, pooled_v, sems):
    wid = lax.axis_index("s") * info.num_cores + lax.axis_index("c")
    base = wid * b_per_w

    pltpu.sync_copy(x_hbm.at[pl.ds(base * CHUNKS_PER_ROW, n_chunks)], idx_v)

    def fire(b, p):
      pltpu.async_copy(
          table_hbm.at[idx_v.at[b * 2]], rows_v.at[p, pl.ds(0, CHUNK)],
          sems[p])
      pltpu.async_copy(
          table_hbm.at[idx_v.at[b * 2 + 1]], rows_v.at[p, pl.ds(CHUNK, CHUNK)],
          sems[p])

    def consume(b, p):
      pltpu.make_async_copy(
          table_hbm.at[pl.ds(0, HIST)], rows_v.at[p], sems[p]).wait()

      def sum_body(i, acc):
        a0, a1, a2, a3 = acc
        l0 = i * UNROLL
        for u in range(UNROLL):
          a0 = a0 + rows_v[p, l0 + u, pl.ds(0, 16)]
          a1 = a1 + rows_v[p, l0 + u, pl.ds(16, 16)]
          a2 = a2 + rows_v[p, l0 + u, pl.ds(32, 16)]
          a3 = a3 + rows_v[p, l0 + u, pl.ds(48, 16)]
        return (a0, a1, a2, a3)

      zero = jnp.zeros((16,), jnp.float32)
      a0, a1, a2, a3 = lax.fori_loop(
          0, HIST // UNROLL, sum_body, (zero, zero, zero, zero))
      pooled_v[b, pl.ds(0, 16)] = a0
      pooled_v[b, pl.ds(16, 16)] = a1
      pooled_v[b, pl.ds(32, 16)] = a2
      pooled_v[b, pl.ds(48, 16)] = a3

    for p in range(NBUF):
      fire(p, p)

    def group_body(g, _):
      for p in range(NBUF):
        b = g * NBUF + p
        consume(b, p)
        fire(b + NBUF, p)
      return ()

    n_groups = b_per_w // NBUF
    lax.fori_loop(0, n_groups - 1, group_body, ())

    for p in range(NBUF):
      consume((n_groups - 1) * NBUF + p, p)

    pltpu.sync_copy(pooled_v, out_hbm.at[pl.ds(base, b_per_w)])

  return pool


_pooling_kernel = _make_pooling_kernel()


def _mlp_kernel(pooled_ref, w1_ref, b1_ref, w2_ref, b2_ref, out_ref):
  h = jnp.dot(pooled_ref[...], w1_ref[...],
              preferred_element_type=jnp.float32)
  h = jnp.maximum(h + b1_ref[...], 0.0)
  out_ref[...] = jnp.dot(h, w2_ref[...],
                         preferred_element_type=jnp.float32) + b2_ref[...]


@jax.jit
def kernel(x, table, W1, b1, W2, b2):
  x2 = x.reshape(BATCH * CHUNKS_PER_ROW, CHUNK)
  pooled = _pooling_kernel(x2, table)
  out = pl.pallas_call(
      _mlp_kernel,
      out_shape=jax.ShapeDtypeStruct((BATCH, 4), jnp.float32),
  )(pooled, W1, b1.reshape(1, 100), W2, b2.reshape(1, 4))
  return out


# speedup vs baseline: 1.2417x; 1.0030x over previous
import functools

import jax
import jax.numpy as jnp
from jax import lax
from jax.experimental import pallas as pl
from jax.experimental.pallas import tpu as pltpu
from jax.experimental.pallas import tpu_sc as plsc

BATCH = 4096
HIST = 200
EMBED = 64
SPLIT = 104

NBUF = 4
UNROLL = 8


def _make_pooling_kernel():
  info = plsc.get_sparse_core_info()
  nw = info.num_cores * info.num_subcores
  b_per_w = BATCH // nw

  mesh = plsc.VectorSubcoreMesh(core_axis_name="c", subcore_axis_name="s")

  @functools.partial(
      pl.kernel,
      mesh=mesh,
      compiler_params=pltpu.CompilerParams(use_tc_tiling_on_sc=False),
      out_type=jax.ShapeDtypeStruct((BATCH, EMBED), jnp.float32),
      scratch_types=[
          pltpu.VMEM((b_per_w, HIST), jnp.int32),
          pltpu.VMEM((NBUF, HIST, EMBED), jnp.float32),
          pltpu.VMEM((b_per_w, EMBED), jnp.float32),
          [pltpu.SemaphoreType.DMA] * NBUF,
      ],
  )
  def pool(x_hbm, table_hbm, out_hbm, idx_v, rows_v, pooled_v, sems):
    wid = lax.axis_index("s") * info.num_cores + lax.axis_index("c")
    base = wid * b_per_w

    pltpu.sync_copy(x_hbm.at[pl.ds(base, b_per_w)], idx_v)

    def fire(b, p):
      pltpu.async_copy(
          table_hbm.at[idx_v.at[b, pl.ds(0, SPLIT)]],
          rows_v.at[p, pl.ds(0, SPLIT)], sems[p])
      pltpu.async_copy(
          table_hbm.at[idx_v.at[b, pl.ds(SPLIT, HIST - SPLIT)]],
          rows_v.at[p, pl.ds(SPLIT, HIST - SPLIT)], sems[p])

    def consume(b, p):
      pltpu.make_async_copy(
          table_hbm.at[pl.ds(0, HIST)], rows_v.at[p], sems[p]).wait()

      def sum_body(i, acc):
        a0, a1, a2, a3 = acc
        l0 = i * UNROLL
        for u in range(UNROLL):
          a0 = a0 + rows_v[p, l0 + u, pl.ds(0, 16)]
          a1 = a1 + rows_v[p, l0 + u, pl.ds(16, 16)]
          a2 = a2 + rows_v[p, l0 + u, pl.ds(32, 16)]
          a3 = a3 + rows_v[p, l0 + u, pl.ds(48, 16)]
        return (a0, a1, a2, a3)

      zero = jnp.zeros((16,), jnp.float32)
      a0, a1, a2, a3 = lax.fori_loop(
          0, HIST // UNROLL, sum_body, (zero, zero, zero, zero))
      pooled_v[b, pl.ds(0, 16)] = a0
      pooled_v[b, pl.ds(16, 16)] = a1
      pooled_v[b, pl.ds(32, 16)] = a2
      pooled_v[b, pl.ds(48, 16)] = a3

    for p in range(NBUF):
      fire(p, p)

    def group_body(g, _):
      for p in range(NBUF):
        b = g * NBUF + p
        consume(b, p)
        fire(b + NBUF, p)
      return ()

    n_groups = b_per_w // NBUF
    lax.fori_loop(0, n_groups - 1, group_body, ())

    for p in range(NBUF):
      consume((n_groups - 1) * NBUF + p, p)

    pltpu.sync_copy(pooled_v, out_hbm.at[pl.ds(base, b_per_w)])

  return pool


_pooling_kernel = _make_pooling_kernel()


def _mlp_kernel(pooled_ref, w1_ref, b1_ref, w2_ref, b2_ref, out_ref):
  h = jnp.dot(pooled_ref[...], w1_ref[...],
              preferred_element_type=jnp.float32)
  h = jnp.maximum(h + b1_ref[...], 0.0)
  out_ref[...] = jnp.dot(h, w2_ref[...],
                         preferred_element_type=jnp.float32) + b2_ref[...]


@jax.jit
def kernel(x, table, W1, b1, W2, b2):
  pooled = _pooling_kernel(x, table)
  out = pl.pallas_call(
      _mlp_kernel,
      out_shape=jax.ShapeDtypeStruct((BATCH, 4), jnp.float32),
  )(pooled, W1, b1.reshape(1, 100), W2, b2.reshape(1, 4))
  return out
